# baseline (speedup 1.0000x reference)
import functools

import jax
import jax.numpy as jnp
from jax.experimental import pallas as pl
from jax.experimental.pallas import tpu as pltpu

_EPS = 1e-8


def _globe_kernel(pp_ref, spt_ref, snt_ref, at_ref, sct_ref, svxt_ref,
                  svyt_ref, svzt_ref, rl_ref, w1_ref, b1_ref, w2_ref, b2_ref,
                  w3_ref, b3_ref, w4_ref, b4_ref, scale_ref, bias_ref,
                  out_ref, *, t_b, s_b, n_s_steps):
    j = pl.program_id(1)
    f32 = jnp.float32

    px = pp_ref[:, 0:1]
    py = pp_ref[:, 1:2]
    pz = pp_ref[:, 2:3]
    sx = spt_ref[0:1, :]
    sy = spt_ref[1:2, :]
    sz = spt_ref[2:3, :]
    rx = px - sx
    ry = py - sy
    rz = pz - sz
    d2 = rx * rx + ry * ry + rz * rz + _EPS * _EPS
    d = jnp.sqrt(d2)
    inv_d = 1.0 / d
    rhx = rx * inv_d
    rhy = ry * inv_d
    rhz = rz * inv_d
    decay = 1.0 / (1.0 + d)

    nx = snt_ref[0:1, :]
    ny = snt_ref[1:2, :]
    nz = snt_ref[2:3, :]
    n_inv = jax.lax.rsqrt(nx * nx + ny * ny + nz * nz + _EPS * _EPS)
    c = (rhx * nx + rhy * ny + rhz * nz) * n_inv

    logd = jnp.log(d)
    f0 = logd - jnp.log(rl_ref[0, 0])
    f1 = logd - jnp.log(rl_ref[0, 1])
    ones = jnp.ones_like(c)
    c2 = c * c
    p2 = 1.5 * c2 - 0.5
    p3 = c * (2.5 * c2 - 1.5)

    fs = jnp.concatenate([f0, f1, ones, c, p2, p3], axis=0)
    feat = jnp.transpose(fs.reshape(6, t_b, s_b), (1, 2, 0))
    n = t_b * s_b
    feat = feat.reshape(n, 6)

    featp = jnp.concatenate([feat[:n // 2], feat[n // 2:]], axis=1)
    h = jnp.tanh(jnp.dot(featp, w1_ref[...], preferred_element_type=f32)
                 + b1_ref[...])
    h = jnp.tanh(jnp.dot(h, w2_ref[...], preferred_element_type=f32)
                 + b2_ref[...])
    h = jnp.tanh(jnp.dot(h, w3_ref[...], preferred_element_type=f32)
                 + b3_ref[...])
    koutp = jnp.dot(h, w4_ref[...], preferred_element_type=f32) + b4_ref[...]
    kout = jnp.concatenate([koutp[:, :24], koutp[:, 24:]], axis=0)

    k24 = jnp.transpose(kout.reshape(t_b, s_b, 24), (2, 0, 1))

    a_row = at_ref[0:1, :]

    p_acc = k24[0] * (sct_ref[0:1, :] * a_row)
    for ch in range(1, 12):
        p_acc += k24[ch] * (sct_ref[ch:ch + 1, :] * a_row)
    p_col = jnp.sum(p_acc * decay, axis=1, keepdims=True)

    gax = k24[12] * (svxt_ref[0:1, :] * a_row)
    gay = k24[12] * (svyt_ref[0:1, :] * a_row)
    gaz = k24[12] * (svzt_ref[0:1, :] * a_row)
    gnx = k24[18] * (svxt_ref[0:1, :] * a_row)
    gny = k24[18] * (svyt_ref[0:1, :] * a_row)
    gnz = k24[18] * (svzt_ref[0:1, :] * a_row)
    for jj in range(1, 6):
        avx = svxt_ref[jj:jj + 1, :] * a_row
        avy = svyt_ref[jj:jj + 1, :] * a_row
        avz = svzt_ref[jj:jj + 1, :] * a_row
        gax += k24[12 + jj] * avx
        gay += k24[12 + jj] * avy
        gaz += k24[12 + jj] * avz
        gnx += k24[18 + jj] * avx
        gny += k24[18 + jj] * avy
        gnz += k24[18 + jj] * avz

    w_ts = (rhx * gax + rhy * gay + rhz * gaz) * decay
    vx_col = jnp.sum(w_ts * rhx + gnx * decay, axis=1, keepdims=True)
    vy_col = jnp.sum(w_ts * rhy + gny * decay, axis=1, keepdims=True)
    vz_col = jnp.sum(w_ts * rhz + gnz * decay, axis=1, keepdims=True)

    partial = jnp.concatenate([p_col, vx_col, vy_col, vz_col], axis=1)

    @pl.when(j == 0)
    def _():
        out_ref[...] = partial

    @pl.when(j != 0)
    def _():
        out_ref[...] = out_ref[...] + partial

    @pl.when(j == n_s_steps - 1)
    def _():
        out_ref[...] = out_ref[...] * scale_ref[...] + bias_ref[...]


def kernel(prediction_points, src_points, src_normals, src_areas,
           src_scalars, src_vectors, reference_lengths,
           W1, b1, W2, b2, W3, b3, W4, b4, p_scale, p_bias, v_scale):
    t, _ = prediction_points.shape
    s, _ = src_points.shape
    t_b = 512
    s_b = 128
    n_t = t // t_b
    n_s = s // s_b

    spt = src_points.T
    snt = src_normals.T
    at = src_areas.reshape(1, s)
    sct = src_scalars.T
    svxt = src_vectors[:, :, 0].T
    svyt = src_vectors[:, :, 1].T
    svzt = src_vectors[:, :, 2].T
    rl2 = reference_lengths.reshape(1, 2)

    def blockdiag(w):
        z = jnp.zeros_like(w)
        return jnp.concatenate([jnp.concatenate([w, z], axis=1),
                                jnp.concatenate([z, w], axis=1)], axis=0)

    w1bd = blockdiag(W1)
    w2bd = blockdiag(W2)
    w3bd = blockdiag(W3)
    w4bd = blockdiag(W4)
    b1r = jnp.concatenate([b1, b1]).reshape(1, 128)
    b2r = jnp.concatenate([b2, b2]).reshape(1, 128)
    b3r = jnp.concatenate([b3, b3]).reshape(1, 128)
    b4r = jnp.concatenate([b4, b4]).reshape(1, 48)
    scale_row = jnp.stack([p_scale, v_scale, v_scale, v_scale]).reshape(1, 4)
    zero = jnp.zeros_like(p_bias)
    bias_row = jnp.stack([p_bias, zero, zero, zero]).reshape(1, 4)

    grid = (n_t, n_s)
    full = lambda shape: pl.BlockSpec(shape, lambda i, j: (0,) * len(shape))
    src_spec = lambda rows: pl.BlockSpec((rows, s_b), lambda i, j: (0, j))
    out = pl.pallas_call(
        functools.partial(_globe_kernel, t_b=t_b, s_b=s_b, n_s_steps=n_s),
        grid=grid,
        in_specs=[
            pl.BlockSpec((t_b, 3), lambda i, j: (i, 0)),
            src_spec(3),
            src_spec(3),
            src_spec(1),
            src_spec(12),
            src_spec(6),
            src_spec(6),
            src_spec(6),
            full((1, 2)),
            full((12, 128)), full((1, 128)),
            full((128, 128)), full((1, 128)),
            full((128, 128)), full((1, 128)),
            full((128, 48)), full((1, 48)),
            full((1, 4)), full((1, 4)),
        ],
        out_specs=pl.BlockSpec((t_b, 4), lambda i, j: (i, 0)),
        out_shape=jax.ShapeDtypeStruct((t, 4), jnp.float32),
        compiler_params=pltpu.CompilerParams(
            dimension_semantics=("parallel", "arbitrary")),
    )(prediction_points, spt, snt, at, sct, svxt, svyt, svzt, rl2,
      w1bd, b1r, w2bd, b2r, w3bd, b3r, w4bd, b4r, scale_row, bias_row)
    return out


# speedup vs baseline: 1.3282x; 1.3282x over previous
import functools

import jax
import jax.numpy as jnp
from jax.experimental import pallas as pl
from jax.experimental.pallas import tpu as pltpu

_EPS = 1e-8


def _globe_kernel(pp_ref, spt_ref, snt_ref, at_ref, sct_ref, svxt_ref,
                  svyt_ref, svzt_ref, w1_ref, b1_ref, w2_ref, b2_ref,
                  w3_ref, b3_ref, w4_ref, b4_ref, scale_ref, bias_ref,
                  out_ref, *, t_b, s_b, n_s_steps):
    j = pl.program_id(1)
    f32 = jnp.float32

    px = pp_ref[:, 0:1]
    py = pp_ref[:, 1:2]
    pz = pp_ref[:, 2:3]
    sx = spt_ref[0:1, :]
    sy = spt_ref[1:2, :]
    sz = spt_ref[2:3, :]
    rx = px - sx
    ry = py - sy
    rz = pz - sz
    d2 = rx * rx + ry * ry + rz * rz + _EPS * _EPS
    d = jnp.sqrt(d2)
    inv_d = 1.0 / d
    rhx = rx * inv_d
    rhy = ry * inv_d
    rhz = rz * inv_d
    decay = 1.0 / (1.0 + d)

    nx = snt_ref[0:1, :]
    ny = snt_ref[1:2, :]
    nz = snt_ref[2:3, :]
    n_inv = jax.lax.rsqrt(nx * nx + ny * ny + nz * nz + _EPS * _EPS)
    c = (rhx * nx + rhy * ny + rhz * nz) * n_inv

    logd = jnp.log(d)
    c2 = c * c
    c3 = c2 * c

    fs = jnp.concatenate([logd, c, c2, c3], axis=0)
    feat = jnp.transpose(fs.reshape(4, t_b, s_b), (1, 2, 0))
    n = t_b * s_b
    feat = feat.reshape(n, 4)

    h = jnp.tanh(jnp.dot(feat, w1_ref[...], preferred_element_type=f32)
                 + b1_ref[...])
    h = jnp.tanh(jnp.dot(h, w2_ref[...], preferred_element_type=f32)
                 + b2_ref[...])
    h = jnp.tanh(jnp.dot(h, w3_ref[...], preferred_element_type=f32)
                 + b3_ref[...])
    kout = jnp.dot(h, w4_ref[...], preferred_element_type=f32) + b4_ref[...]

    k24 = jnp.transpose(kout.reshape(t_b, s_b, 24), (2, 0, 1))

    a_row = at_ref[0:1, :]

    p_acc = k24[0] * (sct_ref[0:1, :] * a_row)
    for ch in range(1, 12):
        p_acc += k24[ch] * (sct_ref[ch:ch + 1, :] * a_row)
    p_col = jnp.sum(p_acc * decay, axis=1, keepdims=True)

    gax = k24[12] * (svxt_ref[0:1, :] * a_row)
    gay = k24[12] * (svyt_ref[0:1, :] * a_row)
    gaz = k24[12] * (svzt_ref[0:1, :] * a_row)
    gnx = k24[18] * (svxt_ref[0:1, :] * a_row)
    gny = k24[18] * (svyt_ref[0:1, :] * a_row)
    gnz = k24[18] * (svzt_ref[0:1, :] * a_row)
    for jj in range(1, 6):
        avx = svxt_ref[jj:jj + 1, :] * a_row
        avy = svyt_ref[jj:jj + 1, :] * a_row
        avz = svzt_ref[jj:jj + 1, :] * a_row
        gax += k24[12 + jj] * avx
        gay += k24[12 + jj] * avy
        gaz += k24[12 + jj] * avz
        gnx += k24[18 + jj] * avx
        gny += k24[18 + jj] * avy
        gnz += k24[18 + jj] * avz

    w_ts = (rhx * gax + rhy * gay + rhz * gaz) * decay
    vx_col = jnp.sum(w_ts * rhx + gnx * decay, axis=1, keepdims=True)
    vy_col = jnp.sum(w_ts * rhy + gny * decay, axis=1, keepdims=True)
    vz_col = jnp.sum(w_ts * rhz + gnz * decay, axis=1, keepdims=True)

    partial = jnp.concatenate([p_col, vx_col, vy_col, vz_col], axis=1)

    @pl.when(j == 0)
    def _():
        out_ref[...] = partial

    @pl.when(j != 0)
    def _():
        out_ref[...] = out_ref[...] + partial

    @pl.when(j == n_s_steps - 1)
    def _():
        out_ref[...] = out_ref[...] * scale_ref[...] + bias_ref[...]


def kernel(prediction_points, src_points, src_normals, src_areas,
           src_scalars, src_vectors, reference_lengths,
           W1, b1, W2, b2, W3, b3, W4, b4, p_scale, p_bias, v_scale):
    t, _ = prediction_points.shape
    s, _ = src_points.shape
    t_b = 512
    s_b = 128
    n_t = t // t_b
    n_s = s // s_b

    spt = src_points.T
    snt = src_normals.T
    at = src_areas.reshape(1, s)
    sct = src_scalars.T
    svxt = src_vectors[:, :, 0].T
    svyt = src_vectors[:, :, 1].T
    svzt = src_vectors[:, :, 2].T
    w1eff = jnp.stack([W1[0] + W1[1],
                       W1[3] - 1.5 * W1[5],
                       1.5 * W1[4],
                       2.5 * W1[5]], axis=0)
    logl = jnp.log(reference_lengths)
    b1r = (b1 + W1[2] - logl[0] * W1[0] - logl[1] * W1[1]
           - 0.5 * W1[4]).reshape(1, 64)
    b2r = b2.reshape(1, 64)
    b3r = b3.reshape(1, 64)
    b4r = b4.reshape(1, 24)
    scale_row = jnp.stack([p_scale, v_scale, v_scale, v_scale]).reshape(1, 4)
    zero = jnp.zeros_like(p_bias)
    bias_row = jnp.stack([p_bias, zero, zero, zero]).reshape(1, 4)

    grid = (n_t, n_s)
    full = lambda shape: pl.BlockSpec(shape, lambda i, j: (0,) * len(shape))
    src_spec = lambda rows: pl.BlockSpec((rows, s_b), lambda i, j: (0, j))
    out = pl.pallas_call(
        functools.partial(_globe_kernel, t_b=t_b, s_b=s_b, n_s_steps=n_s),
        grid=grid,
        in_specs=[
            pl.BlockSpec((t_b, 3), lambda i, j: (i, 0)),
            src_spec(3),
            src_spec(3),
            src_spec(1),
            src_spec(12),
            src_spec(6),
            src_spec(6),
            src_spec(6),
            full((4, 64)), full((1, 64)),
            full((64, 64)), full((1, 64)),
            full((64, 64)), full((1, 64)),
            full((64, 24)), full((1, 24)),
            full((1, 4)), full((1, 4)),
        ],
        out_specs=pl.BlockSpec((t_b, 4), lambda i, j: (i, 0)),
        out_shape=jax.ShapeDtypeStruct((t, 4), jnp.float32),
        compiler_params=pltpu.CompilerParams(
            dimension_semantics=("parallel", "arbitrary")),
    )(prediction_points, spt, snt, at, sct, svxt, svyt, svzt,
      w1eff, b1r, W2, b2r, W3, b3r, W4, b4r, scale_row, bias_row)
    return out
